# baseline (speedup 1.0000x reference)
import functools

import jax
import jax.numpy as jnp
from jax import lax
from jax.experimental import pallas as pl
from jax.experimental.pallas import tpu as pltpu
from jax.experimental.pallas import tpu_sc as plsc

BATCH = 16384
D = 32
L = 16
NC, NS = 2, 16
NW = NC * NS
BPW = BATCH // NW

_mesh = plsc.VectorSubcoreMesh(core_axis_name="c", subcore_axis_name="s")


@functools.partial(
    pl.kernel,
    out_type=jax.ShapeDtypeStruct((BATCH, D), jnp.float32),
    mesh=_mesh,
    compiler_params=pltpu.CompilerParams(use_tc_tiling_on_sc=False),
    scratch_types=[
        pltpu.VMEM((BPW,), jnp.int32),
        pltpu.VMEM((BPW,), jnp.int32),
        pltpu.VMEM((BPW, D), jnp.float32),
        pltpu.VMEM((BPW, D), jnp.float32),
        pltpu.SemaphoreType.DMA,
        pltpu.SemaphoreType.DMA,
    ],
)
def _pmf_sc_kernel(uidx_hbm, iidx_hbm, wu_hbm, wi_hbm, out_hbm,
                   uidx_v, iidx_v, urows_v, irows_v, sem_u, sem_i):
    wid = lax.axis_index("s") * NC + lax.axis_index("c")
    base = wid * BPW

    pltpu.sync_copy(uidx_hbm.at[pl.ds(base, BPW)], uidx_v)
    pltpu.sync_copy(iidx_hbm.at[pl.ds(base, BPW)], iidx_v)

    cu = pltpu.async_copy(wu_hbm.at[uidx_v], urows_v, sem_u)
    ci = pltpu.async_copy(wi_hbm.at[iidx_v], irows_v, sem_i)
    cu.wait()
    ci.wait()

    def body(r, carry):
        u0 = urows_v[r, pl.ds(0, L)]
        u1 = urows_v[r, pl.ds(L, L)]
        i0 = irows_v[r, pl.ds(0, L)]
        i1 = irows_v[r, pl.ds(L, L)]
        urows_v[r, pl.ds(0, L)] = u0 * i0
        urows_v[r, pl.ds(L, L)] = u1 * i1
        return carry

    lax.fori_loop(0, BPW, body, 0)

    pltpu.sync_copy(urows_v, out_hbm.at[pl.ds(base, BPW)])


def kernel(user_indices, item_indices, w_user, w_item):
    return _pmf_sc_kernel(user_indices, item_indices, w_user, w_item)


# speedup vs baseline: 1.3665x; 1.3665x over previous
import functools

import jax
import jax.numpy as jnp
from jax import lax
from jax.experimental import pallas as pl
from jax.experimental.pallas import tpu as pltpu
from jax.experimental.pallas import tpu_sc as plsc

N_ROWS = 1000000
BATCH = 16384
D = 32
L = 16
NC, NS = 2, 16
NW = NC * NS
BPW = BATCH // NW
TR, SUB = 4, 8

CW = 512
NFULL = N_ROWS // CW
TAIL_START = NFULL * CW
TAIL_W = N_ROWS - TAIL_START
NB = 62
CAP = 48
HITCAP = 1024

_mesh = plsc.VectorSubcoreMesh(core_axis_name="c", subcore_axis_name="s")
_params = pltpu.CompilerParams(
    use_tc_tiling_on_sc=True, needs_layout_passes=False)


@functools.partial(
    pl.kernel,
    out_type=(
        jax.ShapeDtypeStruct((BATCH, 128), jnp.float32),
        jax.ShapeDtypeStruct((BATCH, 128), jnp.float32),
    ),
    mesh=_mesh,
    compiler_params=_params,
    scratch_types=[
        pltpu.VMEM((BATCH,), jnp.int32),
        pltpu.VMEM((BATCH,), jnp.int32),
        pltpu.VMEM((TR, SUB, CW), jnp.float32),
        pltpu.VMEM((TR, SUB, TAIL_W), jnp.float32),
        pltpu.VMEM((HITCAP,), jnp.int32),
        pltpu.VMEM((HITCAP,), jnp.int32),
        pltpu.VMEM((NB * CAP,), jnp.int32),
        pltpu.VMEM((NB * CAP,), jnp.int32),
        pltpu.VMEM((CAP, 128), jnp.float32),
        pltpu.VMEM((CAP,), jnp.int32),
        pltpu.SemaphoreType.DMA,
    ],
)
def _sweep_kernel(uidx_hbm, iidx_hbm, wu3, wi3, uval_hbm, ival_hbm,
                  uidx_v, iidx_v, chunk_v, tail_v, hitu_v, hitb_v,
                  bu_v, bb_v, stage_v, bid_v, sem):
    wid = lax.axis_index("s") * NC + lax.axis_index("c")
    pltpu.sync_copy(uidx_hbm, uidx_v)
    pltpu.sync_copy(iidx_hbm, iidx_v)

    lanes = lax.iota(jnp.int32, L)
    nk = jnp.where(wid == 0, NB, NB - 1)

    for idx_v, w3, out_hbm in ((uidx_v, wu3, uval_hbm),
                               (iidx_v, wi3, ival_hbm)):
        def filt(i, off):
            u16 = idx_v[pl.ds(i * L, L)]
            b16 = lanes + i * L
            m = ((u16 >> 9) & (NW - 1)) == wid
            plsc.store_compressed(hitu_v.at[pl.ds(off, L)], u16, mask=m)
            plsc.store_compressed(hitb_v.at[pl.ds(off, L)], b16, mask=m)
            cnt = plsc.all_reduce_population_count(m)
            return off + cnt[0]

        nhit = lax.fori_loop(0, BATCH // L, filt, 0)
        nv = (nhit + L - 1) >> 4

        def prefill(kk, carry):
            safe_u = (kk * NW + wid) << 9
            for t in range(CAP // L):
                bu_v[pl.ds(kk * CAP + t * L, L)] = jnp.full((L,), 0,
                                                            jnp.int32) + safe_u
                bb_v[pl.ds(kk * CAP + t * L, L)] = jnp.full((L,), -1,
                                                            jnp.int32)
            return carry

        lax.fori_loop(0, NB, prefill, 0)

        def bucket(kk, carry):
            def scan(vi, off2):
                u16 = hitu_v[pl.ds(vi * L, L)]
                b16 = hitb_v[pl.ds(vi * L, L)]
                valid = (vi * L + lanes) < nhit
                m2 = ((u16 >> 14) == kk) & valid
                plsc.store_compressed(
                    bu_v.at[pl.ds(kk * CAP + off2, L)], u16, mask=m2)
                plsc.store_compressed(
                    bb_v.at[pl.ds(kk * CAP + off2, L)], b16, mask=m2)
                cnt = plsc.all_reduce_population_count(m2)
                return off2 + cnt[0]

            lax.fori_loop(0, nv, scan, 0)
            return carry

        lax.fori_loop(0, NB, bucket, 0)

        def process_bucket(kk, cs, cref):
            for vs in range(CAP // L):
                slot16 = lanes + vs * L
                u16 = bu_v[pl.ds(kk * CAP + vs * L, L)]
                b16 = bb_v[pl.ds(kk * CAP + vs * L, L)]
                urel = u16 - cs
                bid_v[pl.ds(vs * L, L)] = b16
                for tr in range(TR):
                    for s in range(SUB):
                        d = tr * SUB + s
                        svec = jnp.full((L,), s, jnp.int32)
                        vals = plsc.load_gather(cref.at[tr], [svec, urel])
                        plsc.store_scatter(
                            stage_v, [slot16, jnp.full((L,), d, jnp.int32)],
                            vals)
            pltpu.async_copy(
                stage_v, out_hbm.at[plsc.Indices(bid_v, ignored_value=-1)],
                sem).wait()

        def sweep(j, carry):
            cs = (wid + NW * j) << 9
            cs_al = pl.multiple_of(cs, 128)
            for tr in range(TR):
                pltpu.sync_copy(w3.at[tr, :, pl.ds(cs_al, CW)],
                                chunk_v.at[tr])
            process_bucket(j, cs, chunk_v)
            return carry

        lax.fori_loop(0, nk, sweep, 0)

        @pl.when(wid == 1)
        def _tail():
            for tr in range(TR):
                pltpu.sync_copy(w3.at[tr, :, pl.ds(TAIL_START, TAIL_W)],
                                tail_v.at[tr])
            process_bucket(NB - 1, TAIL_START, tail_v)


@functools.partial(
    pl.kernel,
    out_type=jax.ShapeDtypeStruct((BATCH * D,), jnp.float32),
    mesh=_mesh,
    compiler_params=_params,
    scratch_types=[
        pltpu.VMEM((128, 128), jnp.float32),
        pltpu.VMEM((128, 128), jnp.float32),
        pltpu.VMEM((128 * D,), jnp.float32),
    ],
)
def _mul_kernel(uval_hbm, ival_hbm, out_hbm, u_v, i_v, o_v):
    wid = lax.axis_index("s") * NC + lax.axis_index("c")
    base = wid * BPW

    for sb in range(BPW // 128):
        r0 = base + sb * 128
        pltpu.sync_copy(uval_hbm.at[pl.ds(r0, 128)], u_v)
        pltpu.sync_copy(ival_hbm.at[pl.ds(r0, 128)], i_v)

        def body(r, carry):
            o_v[pl.ds(r * D, L)] = u_v[r, pl.ds(0, L)] * i_v[r, pl.ds(0, L)]
            o_v[pl.ds(r * D + L, L)] = (u_v[r, pl.ds(L, L)]
                                        * i_v[r, pl.ds(L, L)])
            return carry

        lax.fori_loop(0, 128, body, 0)
        pltpu.sync_copy(o_v, out_hbm.at[pl.ds(r0 * D, 128 * D)])


def kernel(user_indices, item_indices, w_user, w_item):
    wu3 = w_user.T.reshape(TR, SUB, N_ROWS)
    wi3 = w_item.T.reshape(TR, SUB, N_ROWS)
    uval, ival = _sweep_kernel(user_indices, item_indices, wu3, wi3)
    flat = _mul_kernel(uval, ival)
    return flat.reshape(BATCH, D)


# speedup vs baseline: 3.0268x; 2.2150x over previous
import functools

import jax
import jax.numpy as jnp
from jax import lax
from jax.experimental import pallas as pl
from jax.experimental.pallas import tpu as pltpu
from jax.experimental.pallas import tpu_sc as plsc

N_ROWS = 1000000
BATCH = 16384
D = 32
L = 16
NC, NS = 2, 16
NW = NC * NS
BPW = BATCH // NW
TR, SUB = 4, 8

CW = 512
NFULL = N_ROWS // CW
TAIL_START = NFULL * CW
TAIL_W = N_ROWS - TAIL_START
NB = 62
CAP = 48
HITCAP = 1024

_mesh = plsc.VectorSubcoreMesh(core_axis_name="c", subcore_axis_name="s")
_params = pltpu.CompilerParams(
    use_tc_tiling_on_sc=True, needs_layout_passes=False)


@functools.partial(
    pl.kernel,
    out_type=(
        jax.ShapeDtypeStruct((BATCH, 128), jnp.float32),
        jax.ShapeDtypeStruct((BATCH, 128), jnp.float32),
    ),
    mesh=_mesh,
    compiler_params=_params,
    scratch_types=[
        pltpu.VMEM((BATCH,), jnp.int32),
        pltpu.VMEM((BATCH,), jnp.int32),
        pltpu.VMEM((TR, SUB, CW), jnp.float32),
        pltpu.VMEM((TR, SUB, CW), jnp.float32),
        pltpu.VMEM((TR, SUB, TAIL_W), jnp.float32),
        pltpu.VMEM((HITCAP,), jnp.int32),
        pltpu.VMEM((HITCAP,), jnp.int32),
        pltpu.VMEM((NB * CAP,), jnp.int32),
        pltpu.VMEM((NB * CAP,), jnp.int32),
        pltpu.VMEM((CAP, 128), jnp.float32),
        pltpu.VMEM((CAP,), jnp.int32),
        pltpu.SemaphoreType.DMA,
        pltpu.SemaphoreType.DMA,
        pltpu.SemaphoreType.DMA,
    ],
)
def _sweep_kernel(uidx_hbm, iidx_hbm, wu3, wi3, uval_hbm, ival_hbm,
                  uidx_v, iidx_v, chunk0_v, chunk1_v, tail_v, hitu_v, hitb_v,
                  bu_v, bb_v, stage_v, bid_v, sem, sem0, sem1):
    wid = lax.axis_index("s") * NC + lax.axis_index("c")
    pltpu.sync_copy(uidx_hbm, uidx_v)
    pltpu.sync_copy(iidx_hbm, iidx_v)

    lanes = lax.iota(jnp.int32, L)
    nk = jnp.where(wid == 0, NB, NB - 1)

    for idx_v, w3, out_hbm in ((uidx_v, wu3, uval_hbm),
                               (iidx_v, wi3, ival_hbm)):
        def filt(i, off):
            u16 = idx_v[pl.ds(i * L, L)]
            b16 = lanes + i * L
            m = ((u16 >> 9) & (NW - 1)) == wid
            plsc.store_compressed(hitu_v.at[pl.ds(off, L)], u16, mask=m)
            plsc.store_compressed(hitb_v.at[pl.ds(off, L)], b16, mask=m)
            cnt = plsc.all_reduce_population_count(m)
            return off + cnt[0]

        nhit = lax.fori_loop(0, BATCH // L, filt, 0)
        nv = (nhit + L - 1) >> 4

        def prefill(kk, carry):
            safe_u = (kk * NW + wid) << 9
            for t in range(CAP // L):
                bu_v[pl.ds(kk * CAP + t * L, L)] = jnp.full((L,), 0,
                                                            jnp.int32) + safe_u
                bb_v[pl.ds(kk * CAP + t * L, L)] = jnp.full((L,), -1,
                                                            jnp.int32)
            return carry

        lax.fori_loop(0, NB, prefill, 0)

        def bucket(kk, carry):
            def scan(vi, off2):
                u16 = hitu_v[pl.ds(vi * L, L)]
                b16 = hitb_v[pl.ds(vi * L, L)]
                valid = (vi * L + lanes) < nhit
                m2 = ((u16 >> 14) == kk) & valid
                plsc.store_compressed(
                    bu_v.at[pl.ds(kk * CAP + off2, L)], u16, mask=m2)
                plsc.store_compressed(
                    bb_v.at[pl.ds(kk * CAP + off2, L)], b16, mask=m2)
                cnt = plsc.all_reduce_population_count(m2)
                return off2 + cnt[0]

            lax.fori_loop(0, nv, scan, 0)
            return carry

        lax.fori_loop(0, NB, bucket, 0)

        def process_bucket(kk, cs, cref):
            for vs in range(CAP // L):
                slot16 = lanes + vs * L
                u16 = bu_v[pl.ds(kk * CAP + vs * L, L)]
                b16 = bb_v[pl.ds(kk * CAP + vs * L, L)]
                urel = u16 - cs
                bid_v[pl.ds(vs * L, L)] = b16
                for tr in range(TR):
                    for s in range(SUB):
                        d = tr * SUB + s
                        svec = jnp.full((L,), s, jnp.int32)
                        vals = plsc.load_gather(cref.at[tr], [svec, urel])
                        plsc.store_scatter(
                            stage_v, [slot16, jnp.full((L,), d, jnp.int32)],
                            vals)
            pltpu.async_copy(
                stage_v, out_hbm.at[plsc.Indices(bid_v, ignored_value=-1)],
                sem).wait()

        def chunk_start(j):
            return pl.multiple_of((wid + NW * j) << 9, 128)

        def issue(j, buf, s):
            pltpu.async_copy(w3.at[:, :, pl.ds(chunk_start(j), CW)], buf, s)

        def drain(buf, s):
            pltpu.make_async_copy(w3.at[:, :, pl.ds(0, CW)], buf, s).wait()

        issue(0, chunk0_v, sem0)

        def sweep_pair(j2, carry):
            j0 = 2 * j2
            j1c = jnp.minimum(2 * j2 + 1, nk - 1)
            jn = jnp.minimum(j0 + 2, nk - 1)
            issue(j1c, chunk1_v, sem1)
            drain(chunk0_v, sem0)
            process_bucket(j0, chunk_start(j0), chunk0_v)
            issue(jn, chunk0_v, sem0)
            drain(chunk1_v, sem1)
            process_bucket(j1c, chunk_start(j1c), chunk1_v)
            return carry

        lax.fori_loop(0, NB // 2, sweep_pair, 0)
        drain(chunk0_v, sem0)

        @pl.when(wid == 1)
        def _tail():
            for tr in range(TR):
                pltpu.sync_copy(w3.at[tr, :, pl.ds(TAIL_START, TAIL_W)],
                                tail_v.at[tr])
            process_bucket(NB - 1, TAIL_START, tail_v)


@functools.partial(
    pl.kernel,
    out_type=jax.ShapeDtypeStruct((BATCH * D,), jnp.float32),
    mesh=_mesh,
    compiler_params=_params,
    scratch_types=[
        pltpu.VMEM((128, 128), jnp.float32),
        pltpu.VMEM((128, 128), jnp.float32),
        pltpu.VMEM((128 * D,), jnp.float32),
    ],
)
def _mul_kernel(uval_hbm, ival_hbm, out_hbm, u_v, i_v, o_v):
    wid = lax.axis_index("s") * NC + lax.axis_index("c")
    base = wid * BPW

    for sb in range(BPW // 128):
        r0 = base + sb * 128
        pltpu.sync_copy(uval_hbm.at[pl.ds(r0, 128)], u_v)
        pltpu.sync_copy(ival_hbm.at[pl.ds(r0, 128)], i_v)

        def body(r, carry):
            o_v[pl.ds(r * D, L)] = u_v[r, pl.ds(0, L)] * i_v[r, pl.ds(0, L)]
            o_v[pl.ds(r * D + L, L)] = (u_v[r, pl.ds(L, L)]
                                        * i_v[r, pl.ds(L, L)])
            return carry

        lax.fori_loop(0, 128, body, 0)
        pltpu.sync_copy(o_v, out_hbm.at[pl.ds(r0 * D, 128 * D)])


def kernel(user_indices, item_indices, w_user, w_item):
    wu3 = w_user.T.reshape(TR, SUB, N_ROWS)
    wi3 = w_item.T.reshape(TR, SUB, N_ROWS)
    uval, ival = _sweep_kernel(user_indices, item_indices, wu3, wi3)
    flat = _mul_kernel(uval, ival)
    return flat.reshape(BATCH, D)


# speedup vs baseline: 3.0299x; 1.0010x over previous
import functools

import jax
import jax.numpy as jnp
from jax import lax
from jax.experimental import pallas as pl
from jax.experimental.pallas import tpu as pltpu
from jax.experimental.pallas import tpu_sc as plsc

N_ROWS = 1000000
BATCH = 16384
D = 32
L = 16
NC, NS = 2, 16
NW = NC * NS
BPW = BATCH // NW
TR, SUB = 4, 8

CW = 512
NFULL = N_ROWS // CW
TAIL_START = NFULL * CW
TAIL_W = N_ROWS - TAIL_START
NB = 62
CAP = 48
HITCAP = 1024

_mesh = plsc.VectorSubcoreMesh(core_axis_name="c", subcore_axis_name="s")
_params = pltpu.CompilerParams(
    use_tc_tiling_on_sc=True, needs_layout_passes=False)


@functools.partial(
    pl.kernel,
    out_type=(
        jax.ShapeDtypeStruct((BATCH, 128), jnp.float32),
        jax.ShapeDtypeStruct((BATCH, 128), jnp.float32),
    ),
    mesh=_mesh,
    compiler_params=_params,
    scratch_types=[
        pltpu.VMEM((BATCH,), jnp.int32),
        pltpu.VMEM((BATCH,), jnp.int32),
        pltpu.VMEM((TR, SUB, CW), jnp.float32),
        pltpu.VMEM((TR, SUB, CW), jnp.float32),
        pltpu.VMEM((TR, SUB, CW), jnp.float32),
        pltpu.VMEM((TR, SUB, CW), jnp.float32),
        pltpu.VMEM((TR, SUB, TAIL_W), jnp.float32),
        pltpu.VMEM((HITCAP,), jnp.int32),
        pltpu.VMEM((HITCAP,), jnp.int32),
        pltpu.VMEM((NB * CAP,), jnp.int32),
        pltpu.VMEM((NB * CAP,), jnp.int32),
        pltpu.VMEM((CAP, 128), jnp.float32),
        pltpu.VMEM((CAP,), jnp.int32),
        pltpu.SemaphoreType.DMA,
        pltpu.SemaphoreType.DMA,
        pltpu.SemaphoreType.DMA,
        pltpu.SemaphoreType.DMA,
        pltpu.SemaphoreType.DMA,
    ],
)
def _sweep_kernel(uidx_hbm, iidx_hbm, wu3, wi3, uval_hbm, ival_hbm,
                  uidx_v, iidx_v, chunk0_v, chunk1_v, chunk2_v, chunk3_v,
                  tail_v, hitu_v, hitb_v,
                  bu_v, bb_v, stage_v, bid_v, sem, sem0, sem1, sem2, sem3):
    wid = lax.axis_index("s") * NC + lax.axis_index("c")
    pltpu.sync_copy(uidx_hbm, uidx_v)
    pltpu.sync_copy(iidx_hbm, iidx_v)

    lanes = lax.iota(jnp.int32, L)
    nk = jnp.where(wid == 0, NB, NB - 1)

    for idx_v, w3, out_hbm in ((uidx_v, wu3, uval_hbm),
                               (iidx_v, wi3, ival_hbm)):
        def filt(i, off):
            u16 = idx_v[pl.ds(i * L, L)]
            b16 = lanes + i * L
            m = ((u16 >> 9) & (NW - 1)) == wid
            plsc.store_compressed(hitu_v.at[pl.ds(off, L)], u16, mask=m)
            plsc.store_compressed(hitb_v.at[pl.ds(off, L)], b16, mask=m)
            cnt = plsc.all_reduce_population_count(m)
            return off + cnt[0]

        nhit = lax.fori_loop(0, BATCH // L, filt, 0)
        nv = (nhit + L - 1) >> 4

        def prefill(kk, carry):
            safe_u = (kk * NW + wid) << 9
            for t in range(CAP // L):
                bu_v[pl.ds(kk * CAP + t * L, L)] = jnp.full((L,), 0,
                                                            jnp.int32) + safe_u
                bb_v[pl.ds(kk * CAP + t * L, L)] = jnp.full((L,), -1,
                                                            jnp.int32)
            return carry

        lax.fori_loop(0, NB, prefill, 0)

        def bucket(kk, carry):
            def scan(vi, off2):
                u16 = hitu_v[pl.ds(vi * L, L)]
                b16 = hitb_v[pl.ds(vi * L, L)]
                valid = (vi * L + lanes) < nhit
                m2 = ((u16 >> 14) == kk) & valid
                plsc.store_compressed(
                    bu_v.at[pl.ds(kk * CAP + off2, L)], u16, mask=m2)
                plsc.store_compressed(
                    bb_v.at[pl.ds(kk * CAP + off2, L)], b16, mask=m2)
                cnt = plsc.all_reduce_population_count(m2)
                return off2 + cnt[0]

            lax.fori_loop(0, nv, scan, 0)
            return carry

        lax.fori_loop(0, NB, bucket, 0)

        def process_bucket(kk, cs, cref):
            for vs in range(CAP // L):
                slot16 = lanes + vs * L
                u16 = bu_v[pl.ds(kk * CAP + vs * L, L)]
                b16 = bb_v[pl.ds(kk * CAP + vs * L, L)]
                urel = u16 - cs
                bid_v[pl.ds(vs * L, L)] = b16
                for tr in range(TR):
                    for s in range(SUB):
                        d = tr * SUB + s
                        svec = jnp.full((L,), s, jnp.int32)
                        vals = plsc.load_gather(cref.at[tr], [svec, urel])
                        plsc.store_scatter(
                            stage_v, [slot16, jnp.full((L,), d, jnp.int32)],
                            vals)
            pltpu.async_copy(
                stage_v, out_hbm.at[plsc.Indices(bid_v, ignored_value=-1)],
                sem).wait()

        def chunk_start(j):
            return pl.multiple_of((wid + NW * j) << 9, 128)

        def issue(j, buf, s):
            pltpu.async_copy(w3.at[:, :, pl.ds(chunk_start(j), CW)], buf, s)

        def drain(buf, s):
            pltpu.make_async_copy(w3.at[:, :, pl.ds(0, CW)], buf, s).wait()

        ring = ((chunk0_v, sem0), (chunk1_v, sem1),
                (chunk2_v, sem2), (chunk3_v, sem3))

        def clamp(j):
            return jnp.minimum(j, nk - 1)

        for t, (buf, s) in enumerate(ring):
            issue(clamp(t), buf, s)

        def sweep_quad(g, carry):
            for t, (buf, s) in enumerate(ring):
                jc = clamp(4 * g + t)
                drain(buf, s)
                process_bucket(jc, chunk_start(jc), buf)
                issue(clamp(4 * g + t + 4), buf, s)
            return carry

        lax.fori_loop(0, (NB + 3) // 4, sweep_quad, 0)
        for buf, s in ring:
            drain(buf, s)

        @pl.when(wid == 1)
        def _tail():
            for tr in range(TR):
                pltpu.sync_copy(w3.at[tr, :, pl.ds(TAIL_START, TAIL_W)],
                                tail_v.at[tr])
            process_bucket(NB - 1, TAIL_START, tail_v)


@functools.partial(
    pl.kernel,
    out_type=jax.ShapeDtypeStruct((BATCH * D,), jnp.float32),
    mesh=_mesh,
    compiler_params=_params,
    scratch_types=[
        pltpu.VMEM((128, 128), jnp.float32),
        pltpu.VMEM((128, 128), jnp.float32),
        pltpu.VMEM((128 * D,), jnp.float32),
    ],
)
def _mul_kernel(uval_hbm, ival_hbm, out_hbm, u_v, i_v, o_v):
    wid = lax.axis_index("s") * NC + lax.axis_index("c")
    base = wid * BPW

    for sb in range(BPW // 128):
        r0 = base + sb * 128
        pltpu.sync_copy(uval_hbm.at[pl.ds(r0, 128)], u_v)
        pltpu.sync_copy(ival_hbm.at[pl.ds(r0, 128)], i_v)

        def body(r, carry):
            o_v[pl.ds(r * D, L)] = u_v[r, pl.ds(0, L)] * i_v[r, pl.ds(0, L)]
            o_v[pl.ds(r * D + L, L)] = (u_v[r, pl.ds(L, L)]
                                        * i_v[r, pl.ds(L, L)])
            return carry

        lax.fori_loop(0, 128, body, 0)
        pltpu.sync_copy(o_v, out_hbm.at[pl.ds(r0 * D, 128 * D)])


def kernel(user_indices, item_indices, w_user, w_item):
    wu3 = w_user.T.reshape(TR, SUB, N_ROWS)
    wi3 = w_item.T.reshape(TR, SUB, N_ROWS)
    uval, ival = _sweep_kernel(user_indices, item_indices, wu3, wi3)
    flat = _mul_kernel(uval, ival)
    return flat.reshape(BATCH, D)


# speedup vs baseline: 3.2707x; 1.0795x over previous
import functools

import jax
import jax.numpy as jnp
from jax import lax
from jax.experimental import pallas as pl
from jax.experimental.pallas import tpu as pltpu
from jax.experimental.pallas import tpu_sc as plsc

N_ROWS = 1000000
BATCH = 16384
D = 32
L = 16
NC, NS = 2, 16
NW = NC * NS
BPW = BATCH // NW
TR, SUB = 4, 8

CW = 512
NFULL = N_ROWS // CW
TAIL_START = NFULL * CW
TAIL_W = N_ROWS - TAIL_START
NB = 62
CAP = 48
HITCAP = 1024

_mesh = plsc.VectorSubcoreMesh(core_axis_name="c", subcore_axis_name="s")
_params = pltpu.CompilerParams(
    use_tc_tiling_on_sc=True, needs_layout_passes=False)


@functools.partial(
    pl.kernel,
    out_type=(
        jax.ShapeDtypeStruct((BATCH, 128), jnp.float32),
        jax.ShapeDtypeStruct((BATCH, 128), jnp.float32),
    ),
    mesh=_mesh,
    compiler_params=_params,
    scratch_types=[
        pltpu.VMEM((BATCH,), jnp.int32),
        pltpu.VMEM((BATCH,), jnp.int32),
        pltpu.VMEM((TR, SUB, CW), jnp.float32),
        pltpu.VMEM((TR, SUB, CW), jnp.float32),
        pltpu.VMEM((TR, SUB, CW), jnp.float32),
        pltpu.VMEM((TR, SUB, CW), jnp.float32),
        pltpu.VMEM((TR, SUB, TAIL_W), jnp.float32),
        pltpu.VMEM((HITCAP,), jnp.int32),
        pltpu.VMEM((HITCAP,), jnp.int32),
        pltpu.VMEM((NB * CAP,), jnp.int32),
        pltpu.VMEM((NB * CAP,), jnp.int32),
        pltpu.VMEM((8 * 128,), jnp.int32),
        pltpu.VMEM((8 * 128,), jnp.int32),
        pltpu.SMEM((8,), jnp.int32),
        pltpu.VMEM((CAP, 128), jnp.float32),
        pltpu.VMEM((CAP,), jnp.int32),
        pltpu.SemaphoreType.DMA,
        pltpu.SemaphoreType.DMA,
        pltpu.SemaphoreType.DMA,
        pltpu.SemaphoreType.DMA,
        pltpu.SemaphoreType.DMA,
    ],
)
def _sweep_kernel(uidx_hbm, iidx_hbm, wu3, wi3, uval_hbm, ival_hbm,
                  uidx_v, iidx_v, chunk0_v, chunk1_v, chunk2_v, chunk3_v,
                  tail_v, hitu_v, hitb_v,
                  bu_v, bb_v, sbu_v, sbb_v, scnt_s,
                  stage_v, bid_v, sem, sem0, sem1, sem2, sem3):
    wid = lax.axis_index("s") * NC + lax.axis_index("c")
    pltpu.sync_copy(uidx_hbm, uidx_v)
    pltpu.sync_copy(iidx_hbm, iidx_v)

    lanes = lax.iota(jnp.int32, L)
    nk = jnp.where(wid == 0, NB, NB - 1)

    for idx_v, w3, out_hbm in ((uidx_v, wu3, uval_hbm),
                               (iidx_v, wi3, ival_hbm)):
        def filt(i, off):
            u16 = idx_v[pl.ds(i * L, L)]
            b16 = lanes + i * L
            m = ((u16 >> 9) & (NW - 1)) == wid
            plsc.store_compressed(hitu_v.at[pl.ds(off, L)], u16, mask=m)
            plsc.store_compressed(hitb_v.at[pl.ds(off, L)], b16, mask=m)
            cnt = plsc.all_reduce_population_count(m)
            return off + cnt[0]

        nhit = lax.fori_loop(0, BATCH // L, filt, 0)
        nv = (nhit + L - 1) >> 4

        def prefill(kk, carry):
            safe_u = (kk * NW + wid) << 9
            for t in range(CAP // L):
                bu_v[pl.ds(kk * CAP + t * L, L)] = jnp.full((L,), 0,
                                                            jnp.int32) + safe_u
                bb_v[pl.ds(kk * CAP + t * L, L)] = jnp.full((L,), -1,
                                                            jnp.int32)
            return carry

        lax.fori_loop(0, NB, prefill, 0)

        def sbucket(sb, carry):
            def sscan(vi, off2):
                u16 = hitu_v[pl.ds(vi * L, L)]
                b16 = hitb_v[pl.ds(vi * L, L)]
                valid = (vi * L + lanes) < nhit
                m2 = ((u16 >> 17) == sb) & valid
                plsc.store_compressed(
                    sbu_v.at[pl.ds(sb * 128 + off2, L)], u16, mask=m2)
                plsc.store_compressed(
                    sbb_v.at[pl.ds(sb * 128 + off2, L)], b16, mask=m2)
                cnt = plsc.all_reduce_population_count(m2)
                return off2 + cnt[0]

            scnt_s[sb] = lax.fori_loop(0, nv, sscan, 0)
            return carry

        lax.fori_loop(0, 8, sbucket, 0)

        def bucket(kk, carry):
            sb = (kk * NW + wid) >> 8
            ns = scnt_s[sb]
            nv2 = (ns + L - 1) >> 4

            def scan(vi, off2):
                u16 = sbu_v[pl.ds(sb * 128 + vi * L, L)]
                b16 = sbb_v[pl.ds(sb * 128 + vi * L, L)]
                valid = (vi * L + lanes) < ns
                m2 = ((u16 >> 14) == kk) & valid
                plsc.store_compressed(
                    bu_v.at[pl.ds(kk * CAP + off2, L)], u16, mask=m2)
                plsc.store_compressed(
                    bb_v.at[pl.ds(kk * CAP + off2, L)], b16, mask=m2)
                cnt = plsc.all_reduce_population_count(m2)
                return off2 + cnt[0]

            lax.fori_loop(0, nv2, scan, 0)
            return carry

        lax.fori_loop(0, NB, bucket, 0)

        def process_bucket(kk, cs, cref):
            for vs in range(CAP // L):
                slot16 = lanes + vs * L
                u16 = bu_v[pl.ds(kk * CAP + vs * L, L)]
                b16 = bb_v[pl.ds(kk * CAP + vs * L, L)]
                urel = u16 - cs
                bid_v[pl.ds(vs * L, L)] = b16
                for tr in range(TR):
                    for s in range(SUB):
                        d = tr * SUB + s
                        svec = jnp.full((L,), s, jnp.int32)
                        vals = plsc.load_gather(cref.at[tr], [svec, urel])
                        plsc.store_scatter(
                            stage_v, [slot16, jnp.full((L,), d, jnp.int32)],
                            vals)
            pltpu.async_copy(
                stage_v, out_hbm.at[plsc.Indices(bid_v, ignored_value=-1)],
                sem).wait()

        def chunk_start(j):
            return pl.multiple_of((wid + NW * j) << 9, 128)

        def issue(j, buf, s):
            pltpu.async_copy(w3.at[:, :, pl.ds(chunk_start(j), CW)], buf, s)

        def drain(buf, s):
            pltpu.make_async_copy(w3.at[:, :, pl.ds(0, CW)], buf, s).wait()

        ring = ((chunk0_v, sem0), (chunk1_v, sem1),
                (chunk2_v, sem2), (chunk3_v, sem3))

        def clamp(j):
            return jnp.minimum(j, nk - 1)

        for t, (buf, s) in enumerate(ring):
            issue(clamp(t), buf, s)

        def sweep_quad(g, carry):
            for t, (buf, s) in enumerate(ring):
                jc = clamp(4 * g + t)
                drain(buf, s)
                process_bucket(jc, chunk_start(jc), buf)
                issue(clamp(4 * g + t + 4), buf, s)
            return carry

        lax.fori_loop(0, (NB + 3) // 4, sweep_quad, 0)
        for buf, s in ring:
            drain(buf, s)

        @pl.when(wid == 1)
        def _tail():
            for tr in range(TR):
                pltpu.sync_copy(w3.at[tr, :, pl.ds(TAIL_START, TAIL_W)],
                                tail_v.at[tr])
            process_bucket(NB - 1, TAIL_START, tail_v)


@functools.partial(
    pl.kernel,
    out_type=jax.ShapeDtypeStruct((BATCH * D,), jnp.float32),
    mesh=_mesh,
    compiler_params=_params,
    scratch_types=[
        pltpu.VMEM((128, 128), jnp.float32),
        pltpu.VMEM((128, 128), jnp.float32),
        pltpu.VMEM((128 * D,), jnp.float32),
    ],
)
def _mul_kernel(uval_hbm, ival_hbm, out_hbm, u_v, i_v, o_v):
    wid = lax.axis_index("s") * NC + lax.axis_index("c")
    base = wid * BPW

    for sb in range(BPW // 128):
        r0 = base + sb * 128
        pltpu.sync_copy(uval_hbm.at[pl.ds(r0, 128)], u_v)
        pltpu.sync_copy(ival_hbm.at[pl.ds(r0, 128)], i_v)

        def body(r, carry):
            o_v[pl.ds(r * D, L)] = u_v[r, pl.ds(0, L)] * i_v[r, pl.ds(0, L)]
            o_v[pl.ds(r * D + L, L)] = (u_v[r, pl.ds(L, L)]
                                        * i_v[r, pl.ds(L, L)])
            return carry

        lax.fori_loop(0, 128, body, 0)
        pltpu.sync_copy(o_v, out_hbm.at[pl.ds(r0 * D, 128 * D)])


def kernel(user_indices, item_indices, w_user, w_item):
    wu3 = w_user.T.reshape(TR, SUB, N_ROWS)
    wi3 = w_item.T.reshape(TR, SUB, N_ROWS)
    uval, ival = _sweep_kernel(user_indices, item_indices, wu3, wi3)
    flat = _mul_kernel(uval, ival)
    return flat.reshape(BATCH, D)


# speedup vs baseline: 4.3951x; 1.3438x over previous
import functools

import jax
import jax.numpy as jnp
from jax import lax
from jax.experimental import pallas as pl
from jax.experimental.pallas import tpu as pltpu
from jax.experimental.pallas import tpu_sc as plsc

N_ROWS = 1000000
BATCH = 16384
D = 32
L = 16
NC, NS = 2, 16
NW = NC * NS
BPW = BATCH // NW
TR, SUB = 4, 8

CW = 512
NFULL = N_ROWS // CW
TAIL_START = NFULL * CW
TAIL_W = N_ROWS - TAIL_START
NB = 62
CAP = 48
HITCAP = 1024

_mesh = plsc.VectorSubcoreMesh(core_axis_name="c", subcore_axis_name="s")
_params = pltpu.CompilerParams(
    use_tc_tiling_on_sc=True, needs_layout_passes=False)


@functools.partial(
    pl.kernel,
    out_type=(
        jax.ShapeDtypeStruct((BATCH, 128), jnp.float32),
        jax.ShapeDtypeStruct((BATCH, 128), jnp.float32),
    ),
    mesh=_mesh,
    compiler_params=_params,
    scratch_types=[
        pltpu.VMEM((BATCH,), jnp.int32),
        pltpu.VMEM((BATCH,), jnp.int32),
        pltpu.VMEM((TR, SUB, CW), jnp.float32),
        pltpu.VMEM((TR, SUB, CW), jnp.float32),
        pltpu.VMEM((TR, SUB, CW), jnp.float32),
        pltpu.VMEM((TR, SUB, CW), jnp.float32),
        pltpu.VMEM((TR, SUB, TAIL_W), jnp.float32),
        pltpu.VMEM((HITCAP,), jnp.int32),
        pltpu.VMEM((HITCAP,), jnp.int32),
        pltpu.VMEM((NB * CAP,), jnp.int32),
        pltpu.VMEM((NB * CAP,), jnp.int32),
        pltpu.VMEM((8 * 128,), jnp.int32),
        pltpu.VMEM((8 * 128,), jnp.int32),
        pltpu.SMEM((8,), jnp.int32),
        pltpu.SMEM((NB,), jnp.int32),
        pltpu.VMEM((CAP, 128), jnp.float32),
        pltpu.VMEM((CAP,), jnp.int32),
        pltpu.SemaphoreType.DMA,
        pltpu.SemaphoreType.DMA,
        pltpu.SemaphoreType.DMA,
        pltpu.SemaphoreType.DMA,
        pltpu.SemaphoreType.DMA,
    ],
)
def _sweep_kernel(uidx_hbm, iidx_hbm, wu3, wi3, uval_hbm, ival_hbm,
                  uidx_v, iidx_v, chunk0_v, chunk1_v, chunk2_v, chunk3_v,
                  tail_v, hitu_v, hitb_v,
                  bu_v, bb_v, sbu_v, sbb_v, scnt_s, bcnt_s,
                  stage_v, bid_v, sem, sem0, sem1, sem2, sem3):
    wid = lax.axis_index("s") * NC + lax.axis_index("c")
    pltpu.sync_copy(uidx_hbm, uidx_v)
    pltpu.sync_copy(iidx_hbm, iidx_v)

    lanes = lax.iota(jnp.int32, L)
    nk = jnp.where(wid == 0, NB, NB - 1)

    for idx_v, w3, out_hbm in ((uidx_v, wu3, uval_hbm),
                               (iidx_v, wi3, ival_hbm)):
        def filt(i, off):
            u16 = idx_v[pl.ds(i * L, L)]
            b16 = lanes + i * L
            m = ((u16 >> 9) & (NW - 1)) == wid
            plsc.store_compressed(hitu_v.at[pl.ds(off, L)], u16, mask=m)
            plsc.store_compressed(hitb_v.at[pl.ds(off, L)], b16, mask=m)
            cnt = plsc.all_reduce_population_count(m)
            return off + cnt[0]

        nhit = lax.fori_loop(0, BATCH // L, filt, 0)
        nv = (nhit + L - 1) >> 4

        def prefill(kk, carry):
            safe_u = (kk * NW + wid) << 9
            for t in range(CAP // L):
                bu_v[pl.ds(kk * CAP + t * L, L)] = jnp.full((L,), 0,
                                                            jnp.int32) + safe_u
                bb_v[pl.ds(kk * CAP + t * L, L)] = jnp.full((L,), -1,
                                                            jnp.int32)
            return carry

        lax.fori_loop(0, NB, prefill, 0)

        def sbucket(sb, carry):
            def sscan(vi, off2):
                u16 = hitu_v[pl.ds(vi * L, L)]
                b16 = hitb_v[pl.ds(vi * L, L)]
                valid = (vi * L + lanes) < nhit
                m2 = ((u16 >> 17) == sb) & valid
                plsc.store_compressed(
                    sbu_v.at[pl.ds(sb * 128 + off2, L)], u16, mask=m2)
                plsc.store_compressed(
                    sbb_v.at[pl.ds(sb * 128 + off2, L)], b16, mask=m2)
                cnt = plsc.all_reduce_population_count(m2)
                return off2 + cnt[0]

            scnt_s[sb] = lax.fori_loop(0, nv, sscan, 0)
            return carry

        lax.fori_loop(0, 8, sbucket, 0)

        def bucket(kk, carry):
            sb = (kk * NW + wid) >> 8
            ns = scnt_s[sb]
            nv2 = (ns + L - 1) >> 4

            def scan(vi, off2):
                u16 = sbu_v[pl.ds(sb * 128 + vi * L, L)]
                b16 = sbb_v[pl.ds(sb * 128 + vi * L, L)]
                valid = (vi * L + lanes) < ns
                m2 = ((u16 >> 14) == kk) & valid
                plsc.store_compressed(
                    bu_v.at[pl.ds(kk * CAP + off2, L)], u16, mask=m2)
                plsc.store_compressed(
                    bb_v.at[pl.ds(kk * CAP + off2, L)], b16, mask=m2)
                cnt = plsc.all_reduce_population_count(m2)
                return off2 + cnt[0]

            bcnt_s[kk] = lax.fori_loop(0, nv2, scan, 0)
            return carry

        lax.fori_loop(0, NB, bucket, 0)

        def process_bucket(kk, cs, cref):
            cnt = bcnt_s[kk]

            def do_slot(vs):
                slot16 = lanes + vs * L
                u16 = bu_v[pl.ds(kk * CAP + vs * L, L)]
                b16 = bb_v[pl.ds(kk * CAP + vs * L, L)]
                urel = u16 - cs
                bid_v[pl.ds(vs * L, L)] = b16
                for tr in range(TR):
                    for s in range(SUB):
                        d = tr * SUB + s
                        svec = jnp.full((L,), s, jnp.int32)
                        vals = plsc.load_gather(cref.at[tr], [svec, urel])
                        plsc.store_scatter(
                            stage_v, [slot16, jnp.full((L,), d, jnp.int32)],
                            vals)

            do_slot(0)
            for vs in range(1, CAP // L):
                @pl.when(cnt > vs * L)
                def _full(vs=vs):
                    do_slot(vs)

                @pl.when(cnt <= vs * L)
                def _skip(vs=vs):
                    bid_v[pl.ds(vs * L, L)] = jnp.full((L,), -1, jnp.int32)

            pltpu.async_copy(
                stage_v, out_hbm.at[plsc.Indices(bid_v, ignored_value=-1)],
                sem).wait()

        def chunk_start(j):
            return pl.multiple_of((wid + NW * j) << 9, 128)

        def issue(j, buf, s):
            pltpu.async_copy(w3.at[:, :, pl.ds(chunk_start(j), CW)], buf, s)

        def drain(buf, s):
            pltpu.make_async_copy(w3.at[:, :, pl.ds(0, CW)], buf, s).wait()

        ring = ((chunk0_v, sem0), (chunk1_v, sem1),
                (chunk2_v, sem2), (chunk3_v, sem3))

        def clamp(j):
            return jnp.minimum(j, nk - 1)

        for t, (buf, s) in enumerate(ring):
            issue(clamp(t), buf, s)

        def sweep_quad(g, carry):
            for t, (buf, s) in enumerate(ring):
                jc = clamp(4 * g + t)
                drain(buf, s)
                process_bucket(jc, chunk_start(jc), buf)
                issue(clamp(4 * g + t + 4), buf, s)
            return carry

        lax.fori_loop(0, (NB + 3) // 4, sweep_quad, 0)
        for buf, s in ring:
            drain(buf, s)

        @pl.when(wid == 1)
        def _tail():
            for tr in range(TR):
                pltpu.sync_copy(w3.at[tr, :, pl.ds(TAIL_START, TAIL_W)],
                                tail_v.at[tr])
            process_bucket(NB - 1, TAIL_START, tail_v)


@functools.partial(
    pl.kernel,
    out_type=jax.ShapeDtypeStruct((BATCH * D,), jnp.float32),
    mesh=_mesh,
    compiler_params=_params,
    scratch_types=[
        pltpu.VMEM((128, 128), jnp.float32),
        pltpu.VMEM((128, 128), jnp.float32),
        pltpu.VMEM((128 * D,), jnp.float32),
    ],
)
def _mul_kernel(uval_hbm, ival_hbm, out_hbm, u_v, i_v, o_v):
    wid = lax.axis_index("s") * NC + lax.axis_index("c")
    base = wid * BPW

    for sb in range(BPW // 128):
        r0 = base + sb * 128
        pltpu.sync_copy(uval_hbm.at[pl.ds(r0, 128)], u_v)
        pltpu.sync_copy(ival_hbm.at[pl.ds(r0, 128)], i_v)

        def body(r, carry):
            o_v[pl.ds(r * D, L)] = u_v[r, pl.ds(0, L)] * i_v[r, pl.ds(0, L)]
            o_v[pl.ds(r * D + L, L)] = (u_v[r, pl.ds(L, L)]
                                        * i_v[r, pl.ds(L, L)])
            return carry

        lax.fori_loop(0, 128, body, 0)
        pltpu.sync_copy(o_v, out_hbm.at[pl.ds(r0 * D, 128 * D)])


def kernel(user_indices, item_indices, w_user, w_item):
    wu3 = w_user.T.reshape(TR, SUB, N_ROWS)
    wi3 = w_item.T.reshape(TR, SUB, N_ROWS)
    uval, ival = _sweep_kernel(user_indices, item_indices, wu3, wi3)
    flat = _mul_kernel(uval, ival)
    return flat.reshape(BATCH, D)


# speedup vs baseline: 4.7663x; 1.0845x over previous
import functools

import jax
import jax.numpy as jnp
from jax import lax
from jax.experimental import pallas as pl
from jax.experimental.pallas import tpu as pltpu
from jax.experimental.pallas import tpu_sc as plsc

N_ROWS = 1000000
BATCH = 16384
D = 32
L = 16
NC, NS = 2, 16
NW = NC * NS
BPW = BATCH // NW
TR, SUB = 4, 8

CW = 512
NFULL = N_ROWS // CW
TAIL_START = NFULL * CW
TAIL_W = N_ROWS - TAIL_START
NB = 62
CAP = 48
HITCAP = 1024

_mesh = plsc.VectorSubcoreMesh(core_axis_name="c", subcore_axis_name="s")
_params = pltpu.CompilerParams(
    use_tc_tiling_on_sc=True, needs_layout_passes=False)


@functools.partial(
    pl.kernel,
    out_type=(
        jax.ShapeDtypeStruct((BATCH, 128), jnp.float32),
        jax.ShapeDtypeStruct((BATCH, 128), jnp.float32),
    ),
    mesh=_mesh,
    compiler_params=_params,
    scratch_types=[
        pltpu.VMEM((BATCH,), jnp.int32),
        pltpu.VMEM((BATCH,), jnp.int32),
        pltpu.VMEM((TR, SUB, CW), jnp.float32),
        pltpu.VMEM((TR, SUB, CW), jnp.float32),
        pltpu.VMEM((TR, SUB, CW), jnp.float32),
        pltpu.VMEM((TR, SUB, CW), jnp.float32),
        pltpu.VMEM((TR, SUB, TAIL_W), jnp.float32),
        pltpu.VMEM((HITCAP,), jnp.int32),
        pltpu.VMEM((HITCAP,), jnp.int32),
        pltpu.VMEM((NB * CAP,), jnp.int32),
        pltpu.VMEM((NB * CAP,), jnp.int32),
        pltpu.VMEM((8 * 128,), jnp.int32),
        pltpu.VMEM((8 * 128,), jnp.int32),
        pltpu.SMEM((8,), jnp.int32),
        pltpu.SMEM((NB,), jnp.int32),
        pltpu.VMEM((CAP, 128), jnp.float32),
        pltpu.VMEM((CAP,), jnp.int32),
        pltpu.SemaphoreType.DMA,
        pltpu.SemaphoreType.DMA,
        pltpu.SemaphoreType.DMA,
        pltpu.SemaphoreType.DMA,
        pltpu.SemaphoreType.DMA,
    ],
)
def _sweep_kernel(uidx_hbm, iidx_hbm, wu3, wi3, uval_hbm, ival_hbm,
                  uidx_v, iidx_v, chunk0_v, chunk1_v, chunk2_v, chunk3_v,
                  tail_v, hitu_v, hitb_v,
                  bu_v, bb_v, sbu_v, sbb_v, scnt_s, bcnt_s,
                  stage_v, bid_v, sem, sem0, sem1, sem2, sem3):
    wid = lax.axis_index("s") * NC + lax.axis_index("c")
    pltpu.sync_copy(uidx_hbm, uidx_v)
    pltpu.sync_copy(iidx_hbm, iidx_v)

    lanes = lax.iota(jnp.int32, L)
    nk = jnp.where(wid == 0, NB, NB - 1)

    for idx_v, w3, out_hbm in ((uidx_v, wu3, uval_hbm),
                               (iidx_v, wi3, ival_hbm)):
        def filt(i, off):
            u16 = idx_v[pl.ds(i * L, L)]
            b16 = lanes + i * L
            m = ((u16 >> 9) & (NW - 1)) == wid
            plsc.store_compressed(hitu_v.at[pl.ds(off, L)], u16, mask=m)
            plsc.store_compressed(hitb_v.at[pl.ds(off, L)], b16, mask=m)
            cnt = plsc.all_reduce_population_count(m)
            return off + cnt[0]

        nhit = lax.fori_loop(0, BATCH // L, filt, 0)
        nv = (nhit + L - 1) >> 4

        def prefill(kk, carry):
            safe_u = (kk * NW + wid) << 9
            for t in range(CAP // L):
                bu_v[pl.ds(kk * CAP + t * L, L)] = jnp.full((L,), 0,
                                                            jnp.int32) + safe_u
                bb_v[pl.ds(kk * CAP + t * L, L)] = jnp.full((L,), -1,
                                                            jnp.int32)
            return carry

        lax.fori_loop(0, NB, prefill, 0)

        def sbucket(sb, carry):
            def sscan(vi, off2):
                u16 = hitu_v[pl.ds(vi * L, L)]
                b16 = hitb_v[pl.ds(vi * L, L)]
                valid = (vi * L + lanes) < nhit
                m2 = ((u16 >> 17) == sb) & valid
                plsc.store_compressed(
                    sbu_v.at[pl.ds(sb * 128 + off2, L)], u16, mask=m2)
                plsc.store_compressed(
                    sbb_v.at[pl.ds(sb * 128 + off2, L)], b16, mask=m2)
                cnt = plsc.all_reduce_population_count(m2)
                return off2 + cnt[0]

            scnt_s[sb] = lax.fori_loop(0, nv, sscan, 0)
            return carry

        lax.fori_loop(0, 8, sbucket, 0)

        def bucket(kk, carry):
            sb = (kk * NW + wid) >> 8
            ns = scnt_s[sb]
            nv2 = (ns + L - 1) >> 4

            def scan(vi, off2):
                u16 = sbu_v[pl.ds(sb * 128 + vi * L, L)]
                b16 = sbb_v[pl.ds(sb * 128 + vi * L, L)]
                valid = (vi * L + lanes) < ns
                m2 = ((u16 >> 14) == kk) & valid
                plsc.store_compressed(
                    bu_v.at[pl.ds(kk * CAP + off2, L)], u16, mask=m2)
                plsc.store_compressed(
                    bb_v.at[pl.ds(kk * CAP + off2, L)], b16, mask=m2)
                cnt = plsc.all_reduce_population_count(m2)
                return off2 + cnt[0]

            bcnt_s[kk] = lax.fori_loop(0, nv2, scan, 0)
            return carry

        lax.fori_loop(0, NB, bucket, 0)

        def process_bucket(kk, cs, cref):
            cnt = bcnt_s[kk]

            def do_slot(vs):
                slot16 = lanes + vs * L
                u16 = bu_v[pl.ds(kk * CAP + vs * L, L)]
                b16 = bb_v[pl.ds(kk * CAP + vs * L, L)]
                urel = u16 - cs
                bid_v[pl.ds(vs * L, L)] = b16
                for tr in range(TR):
                    for s in range(SUB):
                        d = tr * SUB + s
                        svec = jnp.full((L,), s, jnp.int32)
                        vals = plsc.load_gather(cref.at[tr], [svec, urel])
                        plsc.store_scatter(
                            stage_v, [slot16, jnp.full((L,), d, jnp.int32)],
                            vals)

            do_slot(0)
            for vs in range(1, CAP // L):
                @pl.when(cnt > vs * L)
                def _full(vs=vs):
                    do_slot(vs)

                @pl.when(cnt <= vs * L)
                def _skip(vs=vs):
                    bid_v[pl.ds(vs * L, L)] = jnp.full((L,), -1, jnp.int32)

            pltpu.async_copy(
                stage_v, out_hbm.at[plsc.Indices(bid_v, ignored_value=-1)],
                sem).wait()

        def chunk_start(j):
            return pl.multiple_of((wid + NW * j) << 9, 128)

        def issue(j, buf, s):
            pltpu.async_copy(w3.at[:, :, pl.ds(chunk_start(j), CW)], buf, s)

        def drain(buf, s):
            pltpu.make_async_copy(w3.at[:, :, pl.ds(0, CW)], buf, s).wait()

        ring = ((chunk0_v, sem0), (chunk1_v, sem1),
                (chunk2_v, sem2), (chunk3_v, sem3))

        def clamp(j):
            return jnp.minimum(j, nk - 1)

        for t, (buf, s) in enumerate(ring):
            issue(clamp(t), buf, s)

        def sweep_quad(g, carry):
            for t, (buf, s) in enumerate(ring):
                jc = clamp(4 * g + t)
                drain(buf, s)
                process_bucket(jc, chunk_start(jc), buf)
                issue(clamp(4 * g + t + 4), buf, s)
            return carry

        lax.fori_loop(0, (NB + 3) // 4, sweep_quad, 0)
        for buf, s in ring:
            drain(buf, s)

        @pl.when(wid == 1)
        def _tail():
            for tr in range(TR):
                pltpu.sync_copy(w3.at[tr, :, pl.ds(TAIL_START, TAIL_W)],
                                tail_v.at[tr])
            process_bucket(NB - 1, TAIL_START, tail_v)


def _mul_tc_body(u_ref, i_ref, o_ref):
    o_ref[...] = u_ref[:, :D] * i_ref[:, :D]


_mul_tc = pl.pallas_call(
    _mul_tc_body,
    out_shape=jax.ShapeDtypeStruct((BATCH, D), jnp.float32),
    grid=(BATCH // 2048,),
    in_specs=[
        pl.BlockSpec((2048, 128), lambda i: (i, 0)),
        pl.BlockSpec((2048, 128), lambda i: (i, 0)),
    ],
    out_specs=pl.BlockSpec((2048, D), lambda i: (i, 0)),
)


def kernel(user_indices, item_indices, w_user, w_item):
    wu3 = w_user.T.reshape(TR, SUB, N_ROWS)
    wi3 = w_item.T.reshape(TR, SUB, N_ROWS)
    uval, ival = _sweep_kernel(user_indices, item_indices, wu3, wi3)
    return _mul_tc(uval, ival)
